# baseline (speedup 1.0000x reference)
import functools

import jax
import jax.numpy as jnp
from jax import lax
from jax.experimental import pallas as pl
from jax.experimental.pallas import tpu as pltpu
from jax.experimental.pallas import tpu_sc as plsc

NUM_CORES = 2
NUM_SUBCORES = 16
NUM_WORKERS = NUM_CORES * NUM_SUBCORES
NBUF = 2


@functools.partial(jax.jit, static_argnums=(2,))
def _gather_rows(token_ids, weights, D):
    N, S = token_ids.shape
    rows_w = N // NUM_WORKERS
    nsteps = S
    assert nsteps % NBUF == 0 and nsteps >= 2 * NBUF
    mesh = plsc.VectorSubcoreMesh(core_axis_name="c", subcore_axis_name="s")

    @functools.partial(
        pl.kernel,
        out_type=jax.ShapeDtypeStruct((N, S, D), jnp.float32),
        mesh=mesh,
        scratch_types=[
            pltpu.VMEM((rows_w, S), jnp.int32),
            pltpu.VMEM((S, rows_w), jnp.int32),
            pltpu.VMEM((NBUF, rows_w, D), jnp.float32),
            [pltpu.SemaphoreType.DMA] * NBUF,
            [pltpu.SemaphoreType.DMA] * NBUF,
        ],
        compiler_params=pltpu.CompilerParams(
            use_tc_tiling_on_sc=False, needs_layout_passes=False),
    )
    def gather_kernel(ids_hbm, table_hbm, out_hbm, idx_v, idx_t, rows_v,
                      gsem, wsem):
        wid = lax.axis_index("s") * NUM_CORES + lax.axis_index("c")
        base = wid * rows_w
        pltpu.sync_copy(ids_hbm.at[pl.ds(base, rows_w), :], idx_v)

        lanes = lax.iota(jnp.int32, 16)

        def transpose_s(s, carry):
            svec = jnp.full((16,), s, jnp.int32)

            def transpose_c(g, carry2):
                c = g * 16
                vals = plsc.load_gather(idx_v, [c + lanes, svec])
                idx_t[s, pl.ds(c, 16)] = vals
                return carry2

            lax.fori_loop(0, rows_w // 16, transpose_c, 0, unroll=4)
            return carry

        lax.fori_loop(0, S, transpose_s, 0)

        def start_gather(g, b):
            pltpu.async_copy(
                table_hbm.at[idx_t.at[g]], rows_v.at[b], gsem[b])

        def start_write(g, b):
            pltpu.async_copy(
                rows_v.at[b], out_hbm.at[pl.ds(base, rows_w), g], wsem[b])

        def wait_write(b):
            pltpu.make_async_copy(
                rows_v.at[b], out_hbm.at[pl.ds(base, rows_w), 0],
                wsem[b]).wait()

        def wait_gather(b):
            pltpu.make_async_copy(
                table_hbm.at[idx_t.at[0]], rows_v.at[b], gsem[b]).wait()

        for b in range(NBUF - 1):
            start_gather(b, b)

        def group(i, carry):
            for j in range(NBUF):
                g = i * NBUF + j
                nslot = (j - 1) % NBUF

                @pl.when((g + NBUF - 1 < nsteps) & (g >= 1))
                def _():
                    wait_write(nslot)

                @pl.when(g + NBUF - 1 < nsteps)
                def _():
                    start_gather(g + NBUF - 1, nslot)

                wait_gather(j)
                start_write(g, j)
            return carry

        lax.fori_loop(0, nsteps // NBUF, group, 0)
        for b in range(NBUF):
            wait_write(b)

    return gather_kernel(token_ids, weights)


def kernel(token_ids, weights):
    if token_ids.dtype != jnp.int32:
        token_ids = token_ids.astype(jnp.int32)
    return _gather_rows(token_ids, weights, weights.shape[1])
